# Initial kernel scaffold; baseline (speedup 1.0000x reference)
import jax
import jax.numpy as jnp
from jax.experimental import pallas as pl


def kernel(feat, target):
    raise NotImplementedError("write your pallas kernel here")


# speedup vs baseline: 4.3826x; 4.3826x over previous
import functools

import jax
import jax.numpy as jnp
from jax import lax
from jax.experimental import pallas as pl
from jax.experimental.pallas import tpu as pltpu

_MARGIN = 0.25
_GAMMA = 64.0
_B = 1024
_N = 100000
_W = 4096
_NBLK = (_N + _W - 1) // _W
_NEG = -1e30


def _body(tgt_ref, feat_ref, out_ref, m_ref, s_ref, tv_ref):
    c = pl.program_id(0)

    @pl.when(c == 0)
    def _init():
        m_ref[...] = jnp.full((_B, 1), _NEG, jnp.float32)
        s_ref[...] = jnp.zeros((_B, 1), jnp.float32)
        tv_ref[...] = jnp.zeros((_B, 1), jnp.float32)

    x = feat_ref[...]
    iota = lax.broadcasted_iota(jnp.int32, (_B, _W), 1)
    tshift = tgt_ref[...] - c * _W
    is_t = iota == tshift
    invalid = iota >= (_N - c * _W)
    fin = jnp.maximum(x + _MARGIN, 0.0) * ((x - _MARGIN) * _GAMMA)
    fin = jnp.where(is_t | invalid, _NEG, fin)
    tv_ref[...] += jnp.sum(jnp.where(is_t, x, 0.0), axis=1, keepdims=True)
    bmax = jnp.max(fin, axis=1, keepdims=True)
    m_old = m_ref[...]
    m_new = jnp.maximum(m_old, bmax)
    s_ref[...] = s_ref[...] * jnp.exp(m_old - m_new) + jnp.sum(
        jnp.exp(fin - m_new), axis=1, keepdims=True
    )
    m_ref[...] = m_new

    @pl.when(c == _NBLK - 1)
    def _fin():
        tv = tv_ref[...]
        ft = jnp.maximum(1.0 + _MARGIN - tv, 0.0) * ((tv - (1.0 - _MARGIN)) * _GAMMA)
        m = m_ref[...]
        s = s_ref[...]
        big = jnp.maximum(m, ft)
        tot = s * jnp.exp(m - big) + jnp.exp(ft - big)
        lse = big + jnp.log(tot)
        out_ref[...] = jnp.mean(lse - ft).reshape(1, 1)


@functools.partial(jax.jit, static_argnames=("interpret",))
def kernel(feat, target, interpret=False):
    tgt = target.astype(jnp.int32).reshape(_B, 1)
    out = pl.pallas_call(
        _body,
        grid=(_NBLK,),
        in_specs=[
            pl.BlockSpec((_B, 1), lambda c: (0, 0)),
            pl.BlockSpec((_B, _W), lambda c: (0, c)),
        ],
        out_specs=pl.BlockSpec((1, 1), lambda c: (0, 0)),
        out_shape=jax.ShapeDtypeStruct((1, 1), jnp.float32),
        scratch_shapes=[
            pltpu.VMEM((_B, 1), jnp.float32),
            pltpu.VMEM((_B, 1), jnp.float32),
            pltpu.VMEM((_B, 1), jnp.float32),
        ],
        interpret=interpret,
    )(tgt, feat)
    return out[0, 0]


# speedup vs baseline: 9.6733x; 2.2072x over previous
import functools

import jax
import jax.numpy as jnp
from jax import lax
from jax.experimental import pallas as pl
from jax.experimental.pallas import tpu as pltpu

_MARGIN = 0.25
_GAMMA = 64.0
_B = 1024
_N = 100000
_H = 2000
_NBLK = _N // _H
_LOG2E = 1.4426950408889634
_LN2 = 0.6931471805599453
_C0 = _GAMMA * _MARGIN * _MARGIN * _LOG2E
_CS = 9.60897927029168
_NEG = -1e30


def _body(tgt_ref, feat_ref, out_ref, m_ref, s_ref, tv_ref):
    c = pl.program_id(0)

    @pl.when(c == 0)
    def _init():
        m_ref[...] = jnp.zeros((1, _B), jnp.float32)
        s_ref[...] = jnp.zeros((1, _B), jnp.float32)
        tv_ref[...] = jnp.zeros((1, _B), jnp.float32)

    x = feat_ref[...]
    iota = lax.broadcasted_iota(jnp.int32, (_H, _B), 0)
    tsh = tgt_ref[...] - c * _H
    is_t = iota == tsh
    y = x * _CS
    h = jnp.where(x >= -_MARGIN, y * y, _C0)
    h = jnp.where(is_t, _NEG, h)
    tv_ref[...] += jnp.sum(jnp.where(is_t, x, 0.0), axis=0, keepdims=True)
    bmax = jnp.max(h, axis=0, keepdims=True)
    m_old = m_ref[...]
    m_new = jnp.maximum(m_old, bmax)
    s_ref[...] = s_ref[...] * jnp.exp2(m_old - m_new) + jnp.sum(
        jnp.exp2(h - m_new), axis=0, keepdims=True
    )
    m_ref[...] = m_new

    @pl.when(c == _NBLK - 1)
    def _fin():
        tv = tv_ref[...]
        fin_t = jnp.maximum(1.0 + _MARGIN - tv, 0.0) * ((tv - (1.0 - _MARGIN)) * _GAMMA)
        h_t = fin_t * _LOG2E + _C0
        m = m_ref[...]
        s = s_ref[...]
        big = jnp.maximum(m, h_t)
        tot = s * jnp.exp2(m - big) + jnp.exp2(h_t - big)
        lse = (big - _C0 + jnp.log2(tot)) * _LN2
        out_ref[...] = jnp.mean(lse - fin_t).reshape(1, 1)


@functools.partial(jax.jit, static_argnames=("interpret",))
def kernel(feat, target, interpret=False):
    tgt = target.astype(jnp.int32).reshape(1, _B)
    feat_t = feat.T
    out = pl.pallas_call(
        _body,
        grid=(_NBLK,),
        in_specs=[
            pl.BlockSpec((1, _B), lambda c: (0, 0)),
            pl.BlockSpec((_H, _B), lambda c: (c, 0)),
        ],
        out_specs=pl.BlockSpec((1, 1), lambda c: (0, 0)),
        out_shape=jax.ShapeDtypeStruct((1, 1), jnp.float32),
        scratch_shapes=[
            pltpu.VMEM((1, _B), jnp.float32),
            pltpu.VMEM((1, _B), jnp.float32),
            pltpu.VMEM((1, _B), jnp.float32),
        ],
        interpret=interpret,
    )(tgt, feat_t)
    return out[0, 0]
